# baseline (speedup 1.0000x reference)
import functools

import jax
import jax.numpy as jnp
from jax import lax
from jax.experimental import pallas as pl
from jax.experimental.pallas import tpu as pltpu
from jax.experimental.pallas import tpu_sc as plsc

N = 10000
R = 8
D = 128
E = 320000

NC, NS = 2, 16
NW = NC * NS
EPW = E // NW
CHUNK = 20
NCHUNK = EPW // CHUNK
SEGS = 25
SEG_CHUNKS = NCHUNK // SEGS
N_PAD = 10240
ROWS_PER_SUB = N_PAD // NS

BN = 1000
NB = N // BN
NBUF = 10


def _gidx_body(t_ref, s_ref, o_ref):
    v = s_ref[...]
    o_ref[...] = (v // BN) * ((R + 1) * BN) + t_ref[...] * BN + v % BN


def _gidx(etype2, src2):
    return pl.pallas_call(
        _gidx_body,
        grid=(1,),
        in_specs=[pl.BlockSpec((2500, 128), lambda i: (0, 0)),
                  pl.BlockSpec((2500, 128), lambda i: (0, 0))],
        out_specs=pl.BlockSpec((2500, 128), lambda i: (0, 0)),
        out_shape=jax.ShapeDtypeStruct((2500, 128), jnp.int32),
    )(etype2, src2)


def _mm_all(x16, w_ref, b_ref, o_ref):
    for r in range(R + 1):
        acc = jnp.dot(x16, w_ref[r], preferred_element_type=jnp.float32)
        if r == R:
            acc = acc + b_ref[...]
        o_ref[pl.ds(r * BN, BN), :] = acc


def _transform1_body(h_ref, w_ref, b_ref, o_ref):
    _mm_all(h_ref[...], w_ref, b_ref, o_ref)


def _transform1(h16, w_all, b):
    return pl.pallas_call(
        _transform1_body,
        grid=(NB,),
        in_specs=[
            pl.BlockSpec((BN, D), lambda n: (n, 0)),
            pl.BlockSpec((R + 1, D, D), lambda n: (0, 0, 0)),
            pl.BlockSpec((1, D), lambda n: (0, 0)),
        ],
        out_specs=pl.BlockSpec(((R + 1) * BN, D), lambda n: (n, 0)),
        out_shape=jax.ShapeDtypeStruct(((R + 1) * N, D), jnp.float32),
        compiler_params=pltpu.CompilerParams(
            dimension_semantics=("parallel",)),
    )(h16, w_all, b)


def _transform2_body(p_ref, tsl_ref, w_ref, b_ref, o_ref):
    x = p_ref[0] + p_ref[1] + tsl_ref[...]
    _mm_all(x.astype(jnp.bfloat16), w_ref, b_ref, o_ref)


def _transform2(parts, t, w_all, b):
    return pl.pallas_call(
        _transform2_body,
        grid=(NB,),
        in_specs=[
            pl.BlockSpec((2, BN, D), lambda n: (0, n, 0)),
            pl.BlockSpec((BN, D), lambda n: (n * (R + 1) + R, 0)),
            pl.BlockSpec((R + 1, D, D), lambda n: (0, 0, 0)),
            pl.BlockSpec((1, D), lambda n: (0, 0)),
        ],
        out_specs=pl.BlockSpec(((R + 1) * BN, D), lambda n: (n, 0)),
        out_shape=jax.ShapeDtypeStruct(((R + 1) * N, D), jnp.float32),
        compiler_params=pltpu.CompilerParams(
            dimension_semantics=("parallel",)),
    )(parts, t, w_all, b)


def _combine_body(p_ref, t_ref, o_ref):
    o_ref[...] = p_ref[0] + p_ref[1] + t_ref[...]


def _combine(parts, t):
    return pl.pallas_call(
        _combine_body,
        grid=(NB,),
        in_specs=[
            pl.BlockSpec((2, BN, D), lambda n: (0, n, 0)),
            pl.BlockSpec((BN, D), lambda n: (n * (R + 1) + R, 0)),
        ],
        out_specs=pl.BlockSpec((BN, D), lambda n: (n, 0)),
        out_shape=jax.ShapeDtypeStruct((N, D), jnp.float32),
        compiler_params=pltpu.CompilerParams(
            dimension_semantics=("parallel",)),
    )(parts, t)


def _scatter(table, gidx3, dst3):
    @functools.partial(
        pl.kernel,
        mesh=plsc.VectorSubcoreMesh(core_axis_name="c", subcore_axis_name="s"),
        out_type=jax.ShapeDtypeStruct((NC, N_PAD, D), jnp.float32),
        scratch_types=[
            pltpu.VMEM((2, SEG_CHUNKS, CHUNK), jnp.int32),
            pltpu.VMEM((2, SEG_CHUNKS, CHUNK), jnp.int32),
            pltpu.VMEM((NBUF, CHUNK, D), jnp.float32),
            pltpu.VMEM_SHARED((N_PAD, D), jnp.float32),
            pltpu.SemaphoreType.DMA((NBUF,)),
            pltpu.SemaphoreType.DMA((NBUF,)),
            pltpu.SemaphoreType.DMA,
        ],
    )
    def k(table_hbm, gidx_hbm, dst_hbm, out_hbm,
          gidx_v, dst_v, rows_v, agg_sh, gsem, ssem, isem):
        cid = lax.axis_index("c")
        sid = lax.axis_index("s")
        wid = sid * NC + cid

        @pl.loop(0, CHUNK)
        def _(i):
            @pl.loop(0, D, step=16)
            def _(c):
                rows_v[0, i, pl.ds(c, 16)] = jnp.zeros((16,), jnp.float32)

        @pl.loop(0, ROWS_PER_SUB, step=CHUNK)
        def _(rr):
            pltpu.sync_copy(
                rows_v.at[0],
                agg_sh.at[pl.ds(sid * ROWS_PER_SUB + rr, CHUNK)])
        plsc.subcore_barrier()

        def _round(j, first, gi, di):
            for b in range(NBUF):
                def _wait_s(b=b):
                    pltpu.make_async_copy(
                        rows_v.at[b], agg_sh.at[di.at[j + b]],
                        ssem.at[b]).wait()
                if first:
                    pass
                else:
                    _wait_s()
                pltpu.async_copy(table_hbm.at[gi.at[j + b]],
                                 rows_v.at[b], gsem.at[b])
            for b in range(NBUF):
                pltpu.make_async_copy(table_hbm.at[gi.at[j + b]],
                                      rows_v.at[b], gsem.at[b]).wait()
                pltpu.async_copy(rows_v.at[b], agg_sh.at[di.at[j + b]],
                                 ssem.at[b], add=True)

        pltpu.async_copy(gidx_hbm.at[wid, 0], gidx_v.at[0], isem)
        pltpu.async_copy(dst_hbm.at[wid, 0], dst_v.at[0], isem)
        for s in range(SEGS):
            gi = gidx_v.at[s % 2]
            di = dst_v.at[s % 2]
            pltpu.make_async_copy(gidx_hbm.at[wid, s], gi, isem).wait()
            pltpu.make_async_copy(dst_hbm.at[wid, s], di, isem).wait()

            _round(0, s == 0, gi, di)
            if s + 1 < SEGS:
                pltpu.async_copy(gidx_hbm.at[wid, s + 1],
                                 gidx_v.at[(s + 1) % 2], isem)
                pltpu.async_copy(dst_hbm.at[wid, s + 1],
                                 dst_v.at[(s + 1) % 2], isem)

            @pl.loop(NBUF, SEG_CHUNKS, step=NBUF)
            def _(j, gi=gi, di=di):
                _round(j, False, gi, di)

        for b in range(NBUF):
            pltpu.make_async_copy(
                rows_v.at[b],
                agg_sh.at[dst_v.at[(SEGS - 1) % 2].at[SEG_CHUNKS - NBUF + b]],
                ssem.at[b]).wait()
        plsc.subcore_barrier()
        pltpu.sync_copy(
            agg_sh.at[pl.ds(sid * ROWS_PER_SUB, ROWS_PER_SUB)],
            out_hbm.at[cid, pl.ds(sid * ROWS_PER_SUB, ROWS_PER_SUB)])

    return k(table, gidx3, dst3)


def kernel(edge_index, edge_type, emb, W1, W1_loop, b1, W2, W2_loop, b2):
    src = edge_index[0].astype(jnp.int32)
    dst = edge_index[1].astype(jnp.int32)
    et = edge_type.astype(jnp.int32)

    gidx = _gidx(et.reshape(2500, 128), src.reshape(2500, 128))
    gidx3 = gidx.reshape(NW, SEGS, SEG_CHUNKS, CHUNK)
    dst3 = dst.reshape(NW, SEGS, SEG_CHUNKS, CHUNK)

    bf16 = jnp.bfloat16
    w1_all = jnp.concatenate([W1, W1_loop[None]], axis=0).astype(bf16)
    w2_all = jnp.concatenate([W2, W2_loop[None]], axis=0).astype(bf16)

    t1 = _transform1(emb.astype(bf16), w1_all, b1.reshape(1, D))
    p1 = _scatter(t1, gidx3, dst3)
    t2 = _transform2(p1, t1, w2_all, b2.reshape(1, D))
    p2 = _scatter(t2, gidx3, dst3)
    x2 = _combine(p2, t2)
    return x2


# speedup vs baseline: 1.1017x; 1.1017x over previous
import functools

import jax
import jax.numpy as jnp
from jax import lax
from jax.experimental import pallas as pl
from jax.experimental.pallas import tpu as pltpu
from jax.experimental.pallas import tpu_sc as plsc

N = 10000
R = 8
D = 128
E = 320000

NC, NS = 2, 16
NW = NC * NS
EPW = E // NW
CHUNK = 40
NCHUNK = EPW // CHUNK
SEGS = 25
SEG_CHUNKS = NCHUNK // SEGS
N_PAD = 10240
ROWS_PER_SUB = N_PAD // NS

BN = 2000
NB = N // BN
NBUF = 5


def _gidx_body(t_ref, s_ref, o_ref):
    v = s_ref[...]
    o_ref[...] = (v // BN) * ((R + 1) * BN) + t_ref[...] * BN + v % BN


def _gidx(etype2, src2):
    return pl.pallas_call(
        _gidx_body,
        grid=(1,),
        in_specs=[pl.BlockSpec((2500, 128), lambda i: (0, 0)),
                  pl.BlockSpec((2500, 128), lambda i: (0, 0))],
        out_specs=pl.BlockSpec((2500, 128), lambda i: (0, 0)),
        out_shape=jax.ShapeDtypeStruct((2500, 128), jnp.int32),
    )(etype2, src2)


def _mm_all(x16, w_ref, b_ref, o_ref):
    for r in range(R + 1):
        acc = jnp.dot(x16, w_ref[r], preferred_element_type=jnp.float32)
        if r == R:
            acc = acc + b_ref[...]
        o_ref[pl.ds(r * BN, BN), :] = acc


def _transform1_body(h_ref, w_ref, b_ref, o_ref):
    _mm_all(h_ref[...], w_ref, b_ref, o_ref)


def _transform1(h16, w_all, b):
    return pl.pallas_call(
        _transform1_body,
        grid=(NB,),
        in_specs=[
            pl.BlockSpec((BN, D), lambda n: (n, 0)),
            pl.BlockSpec((R + 1, D, D), lambda n: (0, 0, 0)),
            pl.BlockSpec((1, D), lambda n: (0, 0)),
        ],
        out_specs=pl.BlockSpec(((R + 1) * BN, D), lambda n: (n, 0)),
        out_shape=jax.ShapeDtypeStruct(((R + 1) * N, D), jnp.float32),
        compiler_params=pltpu.CompilerParams(
            dimension_semantics=("parallel",)),
    )(h16, w_all, b)


def _transform2_body(p_ref, tsl_ref, w_ref, b_ref, o_ref):
    x = p_ref[0] + p_ref[1] + tsl_ref[...]
    _mm_all(x.astype(jnp.bfloat16), w_ref, b_ref, o_ref)


def _transform2(parts, t, w_all, b):
    return pl.pallas_call(
        _transform2_body,
        grid=(NB,),
        in_specs=[
            pl.BlockSpec((2, BN, D), lambda n: (0, n, 0)),
            pl.BlockSpec((BN, D), lambda n: (n * (R + 1) + R, 0)),
            pl.BlockSpec((R + 1, D, D), lambda n: (0, 0, 0)),
            pl.BlockSpec((1, D), lambda n: (0, 0)),
        ],
        out_specs=pl.BlockSpec(((R + 1) * BN, D), lambda n: (n, 0)),
        out_shape=jax.ShapeDtypeStruct(((R + 1) * N, D), jnp.float32),
        compiler_params=pltpu.CompilerParams(
            dimension_semantics=("parallel",)),
    )(parts, t, w_all, b)


def _combine_body(p_ref, t_ref, o_ref):
    o_ref[...] = p_ref[0] + p_ref[1] + t_ref[...]


def _combine(parts, t):
    return pl.pallas_call(
        _combine_body,
        grid=(NB,),
        in_specs=[
            pl.BlockSpec((2, BN, D), lambda n: (0, n, 0)),
            pl.BlockSpec((BN, D), lambda n: (n * (R + 1) + R, 0)),
        ],
        out_specs=pl.BlockSpec((BN, D), lambda n: (n, 0)),
        out_shape=jax.ShapeDtypeStruct((N, D), jnp.float32),
        compiler_params=pltpu.CompilerParams(
            dimension_semantics=("parallel",)),
    )(parts, t)


def _scatter(table, gidx3, dst3):
    @functools.partial(
        pl.kernel,
        mesh=plsc.VectorSubcoreMesh(core_axis_name="c", subcore_axis_name="s"),
        out_type=jax.ShapeDtypeStruct((NC, N_PAD, D), jnp.float32),
        scratch_types=[
            pltpu.VMEM((2, SEG_CHUNKS, CHUNK), jnp.int32),
            pltpu.VMEM((2, SEG_CHUNKS, CHUNK), jnp.int32),
            pltpu.VMEM((NBUF, CHUNK, D), jnp.float32),
            pltpu.VMEM_SHARED((N_PAD, D), jnp.float32),
            pltpu.SemaphoreType.DMA((NBUF,)),
            pltpu.SemaphoreType.DMA((NBUF,)),
            pltpu.SemaphoreType.DMA,
        ],
    )
    def k(table_hbm, gidx_hbm, dst_hbm, out_hbm,
          gidx_v, dst_v, rows_v, agg_sh, gsem, ssem, isem):
        cid = lax.axis_index("c")
        sid = lax.axis_index("s")
        wid = sid * NC + cid

        @pl.loop(0, CHUNK)
        def _(i):
            @pl.loop(0, D, step=16)
            def _(c):
                rows_v[0, i, pl.ds(c, 16)] = jnp.zeros((16,), jnp.float32)

        @pl.loop(0, ROWS_PER_SUB, step=CHUNK)
        def _(rr):
            pltpu.sync_copy(
                rows_v.at[0],
                agg_sh.at[pl.ds(sid * ROWS_PER_SUB + rr, CHUNK)])
        plsc.subcore_barrier()

        def _round(j, first, gi, di):
            for b in range(NBUF):
                def _wait_s(b=b):
                    pltpu.make_async_copy(
                        rows_v.at[b], agg_sh.at[di.at[j + b]],
                        ssem.at[b]).wait()
                if first:
                    pass
                else:
                    _wait_s()
                pltpu.async_copy(table_hbm.at[gi.at[j + b]],
                                 rows_v.at[b], gsem.at[b])
            for b in range(NBUF):
                pltpu.make_async_copy(table_hbm.at[gi.at[j + b]],
                                      rows_v.at[b], gsem.at[b]).wait()
                pltpu.async_copy(rows_v.at[b], agg_sh.at[di.at[j + b]],
                                 ssem.at[b], add=True)

        pltpu.async_copy(gidx_hbm.at[wid, 0], gidx_v.at[0], isem)
        pltpu.async_copy(dst_hbm.at[wid, 0], dst_v.at[0], isem)
        for s in range(SEGS):
            gi = gidx_v.at[s % 2]
            di = dst_v.at[s % 2]
            pltpu.make_async_copy(gidx_hbm.at[wid, s], gi, isem).wait()
            pltpu.make_async_copy(dst_hbm.at[wid, s], di, isem).wait()

            _round(0, s == 0, gi, di)
            if s + 1 < SEGS:
                pltpu.async_copy(gidx_hbm.at[wid, s + 1],
                                 gidx_v.at[(s + 1) % 2], isem)
                pltpu.async_copy(dst_hbm.at[wid, s + 1],
                                 dst_v.at[(s + 1) % 2], isem)

            @pl.loop(NBUF, SEG_CHUNKS, step=NBUF)
            def _(j, gi=gi, di=di):
                _round(j, False, gi, di)

        for b in range(NBUF):
            pltpu.make_async_copy(
                rows_v.at[b],
                agg_sh.at[dst_v.at[(SEGS - 1) % 2].at[SEG_CHUNKS - NBUF + b]],
                ssem.at[b]).wait()
        plsc.subcore_barrier()
        pltpu.sync_copy(
            agg_sh.at[pl.ds(sid * ROWS_PER_SUB, ROWS_PER_SUB)],
            out_hbm.at[cid, pl.ds(sid * ROWS_PER_SUB, ROWS_PER_SUB)])

    return k(table, gidx3, dst3)


def kernel(edge_index, edge_type, emb, W1, W1_loop, b1, W2, W2_loop, b2):
    src = edge_index[0].astype(jnp.int32)
    dst = edge_index[1].astype(jnp.int32)
    et = edge_type.astype(jnp.int32)

    gidx = _gidx(et.reshape(2500, 128), src.reshape(2500, 128))
    gidx3 = gidx.reshape(NW, SEGS, SEG_CHUNKS, CHUNK)
    dst3 = dst.reshape(NW, SEGS, SEG_CHUNKS, CHUNK)

    bf16 = jnp.bfloat16
    w1_all = jnp.concatenate([W1, W1_loop[None]], axis=0).astype(bf16)
    w2_all = jnp.concatenate([W2, W2_loop[None]], axis=0).astype(bf16)

    t1 = _transform1(emb.astype(bf16), w1_all, b1.reshape(1, D))
    p1 = _scatter(t1, gidx3, dst3)
    t2 = _transform2(p1, t1, w2_all, b2.reshape(1, D))
    p2 = _scatter(t2, gidx3, dst3)
    x2 = _combine(p2, t2)
    return x2


# speedup vs baseline: 1.1075x; 1.0053x over previous
import functools

import jax
import jax.numpy as jnp
from jax import lax
from jax.experimental import pallas as pl
from jax.experimental.pallas import tpu as pltpu
from jax.experimental.pallas import tpu_sc as plsc

N = 10000
R = 8
D = 128
E = 320000

NC, NS = 2, 16
NW = NC * NS
EPW = E // NW
CHUNK = 40
NCHUNK = EPW // CHUNK
SEGS = 25
SEG_CHUNKS = NCHUNK // SEGS
N_PAD = 10240
ROWS_PER_SUB = N_PAD // NS

BN = 2000
NB = N // BN
NBUF = 5


def _gidx_body(t_ref, s_ref, o_ref):
    v = s_ref[...]
    o_ref[...] = (v // BN) * ((R + 1) * BN) + t_ref[...] * BN + v % BN


def _gidx(etype2, src2):
    return pl.pallas_call(
        _gidx_body,
        grid=(1,),
        in_specs=[pl.BlockSpec((2500, 128), lambda i: (0, 0)),
                  pl.BlockSpec((2500, 128), lambda i: (0, 0))],
        out_specs=pl.BlockSpec((2500, 128), lambda i: (0, 0)),
        out_shape=jax.ShapeDtypeStruct((2500, 128), jnp.int32),
    )(etype2, src2)


def _mm_all(x16, w_ref, b_ref, o_ref):
    for r in range(R + 1):
        acc = jnp.dot(x16, w_ref[r], preferred_element_type=jnp.float32)
        if r == R:
            acc = acc + b_ref[...]
        o_ref[pl.ds(r * BN, BN), :] = acc


def _transform1_body(h_ref, w_ref, b_ref, o_ref):
    _mm_all(h_ref[...], w_ref, b_ref, o_ref)


def _transform1(h16, w_all, b):
    return pl.pallas_call(
        _transform1_body,
        grid=(NB,),
        in_specs=[
            pl.BlockSpec((BN, D), lambda n: (n, 0)),
            pl.BlockSpec((R + 1, D, D), lambda n: (0, 0, 0)),
            pl.BlockSpec((1, D), lambda n: (0, 0)),
        ],
        out_specs=pl.BlockSpec(((R + 1) * BN, D), lambda n: (n, 0)),
        out_shape=jax.ShapeDtypeStruct(((R + 1) * N, D), jnp.float32),
        compiler_params=pltpu.CompilerParams(
            dimension_semantics=("parallel",)),
    )(h16, w_all, b)


def _transform2_body(p_ref, tsl_ref, w_ref, b_ref, o_ref):
    x = p_ref[0] + p_ref[1] + tsl_ref[...]
    _mm_all(x.astype(jnp.bfloat16), w_ref, b_ref, o_ref)


def _transform2(parts, t, w_all, b):
    return pl.pallas_call(
        _transform2_body,
        grid=(NB,),
        in_specs=[
            pl.BlockSpec((2, BN, D), lambda n: (0, n, 0)),
            pl.BlockSpec((BN, D), lambda n: (n * (R + 1) + R, 0)),
            pl.BlockSpec((R + 1, D, D), lambda n: (0, 0, 0)),
            pl.BlockSpec((1, D), lambda n: (0, 0)),
        ],
        out_specs=pl.BlockSpec(((R + 1) * BN, D), lambda n: (n, 0)),
        out_shape=jax.ShapeDtypeStruct(((R + 1) * N, D), jnp.float32),
        compiler_params=pltpu.CompilerParams(
            dimension_semantics=("parallel",)),
    )(parts, t, w_all, b)


def _combine_body(p_ref, t_ref, o_ref):
    o_ref[...] = p_ref[0] + p_ref[1] + t_ref[...]


def _combine(parts, t):
    return pl.pallas_call(
        _combine_body,
        grid=(NB,),
        in_specs=[
            pl.BlockSpec((2, BN, D), lambda n: (0, n, 0)),
            pl.BlockSpec((BN, D), lambda n: (n * (R + 1) + R, 0)),
        ],
        out_specs=pl.BlockSpec((BN, D), lambda n: (n, 0)),
        out_shape=jax.ShapeDtypeStruct((N, D), jnp.float32),
        compiler_params=pltpu.CompilerParams(
            dimension_semantics=("parallel",)),
    )(parts, t)


def _scatter(table, gidx3, dst3):
    @functools.partial(
        pl.kernel,
        mesh=plsc.VectorSubcoreMesh(core_axis_name="c", subcore_axis_name="s"),
        out_type=jax.ShapeDtypeStruct((NC, N_PAD, D), jnp.float32),
        scratch_types=[
            pltpu.VMEM((2, SEG_CHUNKS, CHUNK), jnp.int32),
            pltpu.VMEM((2, SEG_CHUNKS, CHUNK), jnp.int32),
            pltpu.VMEM((NBUF, CHUNK, D), jnp.float32),
            pltpu.VMEM_SHARED((N_PAD, D), jnp.float32),
            pltpu.SemaphoreType.DMA((NBUF,)),
            pltpu.SemaphoreType.DMA((NBUF,)),
            pltpu.SemaphoreType.DMA,
        ],
    )
    def k(table_hbm, gidx_hbm, dst_hbm, out_hbm,
          gidx_v, dst_v, rows_v, agg_sh, gsem, ssem, isem):
        cid = lax.axis_index("c")
        sid = lax.axis_index("s")
        wid = sid * NC + cid

        @pl.loop(0, CHUNK)
        def _(i):
            @pl.loop(0, D, step=16)
            def _(c):
                rows_v[0, i, pl.ds(c, 16)] = jnp.zeros((16,), jnp.float32)

        for z in range(ROWS_PER_SUB // CHUNK):
            pltpu.async_copy(
                rows_v.at[0],
                agg_sh.at[pl.ds(sid * ROWS_PER_SUB + z * CHUNK, CHUNK)],
                gsem.at[0])
        for z in range(ROWS_PER_SUB // CHUNK):
            pltpu.make_async_copy(
                rows_v.at[0],
                agg_sh.at[pl.ds(sid * ROWS_PER_SUB + z * CHUNK, CHUNK)],
                gsem.at[0]).wait()
        plsc.subcore_barrier()

        def _round(j, first, gi, di):
            for b in range(NBUF):
                def _wait_s(b=b):
                    pltpu.make_async_copy(
                        rows_v.at[b], agg_sh.at[di.at[j + b]],
                        ssem.at[b]).wait()
                if first:
                    pass
                else:
                    _wait_s()
                pltpu.async_copy(table_hbm.at[gi.at[j + b]],
                                 rows_v.at[b], gsem.at[b])
            for b in range(NBUF):
                pltpu.make_async_copy(table_hbm.at[gi.at[j + b]],
                                      rows_v.at[b], gsem.at[b]).wait()
                pltpu.async_copy(rows_v.at[b], agg_sh.at[di.at[j + b]],
                                 ssem.at[b], add=True)

        pltpu.async_copy(gidx_hbm.at[wid, 0], gidx_v.at[0], isem)
        pltpu.async_copy(dst_hbm.at[wid, 0], dst_v.at[0], isem)
        for s in range(SEGS):
            gi = gidx_v.at[s % 2]
            di = dst_v.at[s % 2]
            pltpu.make_async_copy(gidx_hbm.at[wid, s], gi, isem).wait()
            pltpu.make_async_copy(dst_hbm.at[wid, s], di, isem).wait()

            _round(0, s == 0, gi, di)
            if s + 1 < SEGS:
                pltpu.async_copy(gidx_hbm.at[wid, s + 1],
                                 gidx_v.at[(s + 1) % 2], isem)
                pltpu.async_copy(dst_hbm.at[wid, s + 1],
                                 dst_v.at[(s + 1) % 2], isem)

            @pl.loop(NBUF, SEG_CHUNKS, step=NBUF)
            def _(j, gi=gi, di=di):
                _round(j, False, gi, di)

        for b in range(NBUF):
            pltpu.make_async_copy(
                rows_v.at[b],
                agg_sh.at[dst_v.at[(SEGS - 1) % 2].at[SEG_CHUNKS - NBUF + b]],
                ssem.at[b]).wait()
        plsc.subcore_barrier()
        pltpu.sync_copy(
            agg_sh.at[pl.ds(sid * ROWS_PER_SUB, ROWS_PER_SUB)],
            out_hbm.at[cid, pl.ds(sid * ROWS_PER_SUB, ROWS_PER_SUB)])

    return k(table, gidx3, dst3)


def kernel(edge_index, edge_type, emb, W1, W1_loop, b1, W2, W2_loop, b2):
    src = edge_index[0].astype(jnp.int32)
    dst = edge_index[1].astype(jnp.int32)
    et = edge_type.astype(jnp.int32)

    gidx = _gidx(et.reshape(2500, 128), src.reshape(2500, 128))
    gidx3 = gidx.reshape(NW, SEGS, SEG_CHUNKS, CHUNK)
    dst3 = dst.reshape(NW, SEGS, SEG_CHUNKS, CHUNK)

    bf16 = jnp.bfloat16
    w1_all = jnp.concatenate([W1, W1_loop[None]], axis=0).astype(bf16)
    w2_all = jnp.concatenate([W2, W2_loop[None]], axis=0).astype(bf16)

    t1 = _transform1(emb.astype(bf16), w1_all, b1.reshape(1, D))
    p1 = _scatter(t1, gidx3, dst3)
    t2 = _transform2(p1, t1, w2_all, b2.reshape(1, D))
    p2 = _scatter(t2, gidx3, dst3)
    x2 = _combine(p2, t2)
    return x2


# speedup vs baseline: 1.1434x; 1.0325x over previous
import functools

import jax
import jax.numpy as jnp
from jax import lax
from jax.experimental import pallas as pl
from jax.experimental.pallas import tpu as pltpu
from jax.experimental.pallas import tpu_sc as plsc

N = 10000
R = 8
D = 128
E = 320000

NC, NS = 2, 16
NW = NC * NS
EPW = E // NW
CHUNK = 40
NCHUNK = EPW // CHUNK
SEGS = 25
SEG_CHUNKS = NCHUNK // SEGS
N_PAD = 10240
ROWS_PER_SUB = N_PAD // NS

BN = 2000
NB = N // BN
NBUF = 5


def _gidx_body(ei_ref, t_ref, g_ref, d_ref):
    v = ei_ref[0]
    g_ref[...] = (v // BN) * ((R + 1) * BN) + t_ref[...] * BN + v % BN
    d_ref[...] = ei_ref[1]


def _gidx(ei3, etype2):
    return pl.pallas_call(
        _gidx_body,
        grid=(1,),
        in_specs=[pl.BlockSpec((2, 2500, 128), lambda i: (0, 0, 0)),
                  pl.BlockSpec((2500, 128), lambda i: (0, 0))],
        out_specs=[pl.BlockSpec((2500, 128), lambda i: (0, 0)),
                   pl.BlockSpec((2500, 128), lambda i: (0, 0))],
        out_shape=[jax.ShapeDtypeStruct((2500, 128), jnp.int32),
                   jax.ShapeDtypeStruct((2500, 128), jnp.int32)],
    )(ei3, etype2)


def _mm_all(x16, w_ref, b_ref, o_ref):
    for r in range(R + 1):
        acc = jnp.dot(x16, w_ref[r], preferred_element_type=jnp.float32)
        if r == R:
            acc = acc + b_ref[...]
        o_ref[pl.ds(r * BN, BN), :] = acc


def _transform1_body(h_ref, w_ref, b_ref, o_ref):
    _mm_all(h_ref[...], w_ref, b_ref, o_ref)


def _transform1(h16, w_all, b):
    return pl.pallas_call(
        _transform1_body,
        grid=(NB,),
        in_specs=[
            pl.BlockSpec((BN, D), lambda n: (n, 0)),
            pl.BlockSpec((R + 1, D, D), lambda n: (0, 0, 0)),
            pl.BlockSpec((1, D), lambda n: (0, 0)),
        ],
        out_specs=pl.BlockSpec(((R + 1) * BN, D), lambda n: (n, 0)),
        out_shape=jax.ShapeDtypeStruct(((R + 1) * N, D), jnp.float32),
        compiler_params=pltpu.CompilerParams(
            dimension_semantics=("parallel",)),
    )(h16, w_all, b)


def _transform2_body(p_ref, tsl_ref, w_ref, b_ref, o_ref):
    x = p_ref[0] + p_ref[1] + tsl_ref[...]
    _mm_all(x.astype(jnp.bfloat16), w_ref, b_ref, o_ref)


def _transform2(parts, t, w_all, b):
    return pl.pallas_call(
        _transform2_body,
        grid=(NB,),
        in_specs=[
            pl.BlockSpec((2, BN, D), lambda n: (0, n, 0)),
            pl.BlockSpec((BN, D), lambda n: (n * (R + 1) + R, 0)),
            pl.BlockSpec((R + 1, D, D), lambda n: (0, 0, 0)),
            pl.BlockSpec((1, D), lambda n: (0, 0)),
        ],
        out_specs=pl.BlockSpec(((R + 1) * BN, D), lambda n: (n, 0)),
        out_shape=jax.ShapeDtypeStruct(((R + 1) * N, D), jnp.float32),
        compiler_params=pltpu.CompilerParams(
            dimension_semantics=("parallel",)),
    )(parts, t, w_all, b)


def _combine_body(p_ref, t_ref, o_ref):
    o_ref[...] = p_ref[0] + p_ref[1] + t_ref[...]


def _combine(parts, t):
    return pl.pallas_call(
        _combine_body,
        grid=(NB,),
        in_specs=[
            pl.BlockSpec((2, BN, D), lambda n: (0, n, 0)),
            pl.BlockSpec((BN, D), lambda n: (n * (R + 1) + R, 0)),
        ],
        out_specs=pl.BlockSpec((BN, D), lambda n: (n, 0)),
        out_shape=jax.ShapeDtypeStruct((N, D), jnp.float32),
        compiler_params=pltpu.CompilerParams(
            dimension_semantics=("parallel",)),
    )(parts, t)


def _scatter(table, gidx3, dst3):
    @functools.partial(
        pl.kernel,
        mesh=plsc.VectorSubcoreMesh(core_axis_name="c", subcore_axis_name="s"),
        out_type=jax.ShapeDtypeStruct((NC, N_PAD, D), jnp.float32),
        scratch_types=[
            pltpu.VMEM((2, SEG_CHUNKS, CHUNK), jnp.int32),
            pltpu.VMEM((2, SEG_CHUNKS, CHUNK), jnp.int32),
            pltpu.VMEM((NBUF, CHUNK, D), jnp.float32),
            pltpu.VMEM_SHARED((N_PAD, D), jnp.float32),
            pltpu.SemaphoreType.DMA((NBUF,)),
            pltpu.SemaphoreType.DMA((NBUF,)),
            pltpu.SemaphoreType.DMA,
        ],
    )
    def k(table_hbm, gidx_hbm, dst_hbm, out_hbm,
          gidx_v, dst_v, rows_v, agg_sh, gsem, ssem, isem):
        cid = lax.axis_index("c")
        sid = lax.axis_index("s")
        wid = sid * NC + cid

        @pl.loop(0, CHUNK)
        def _(i):
            @pl.loop(0, D, step=16)
            def _(c):
                rows_v[0, i, pl.ds(c, 16)] = jnp.zeros((16,), jnp.float32)

        for z in range(ROWS_PER_SUB // CHUNK):
            pltpu.async_copy(
                rows_v.at[0],
                agg_sh.at[pl.ds(sid * ROWS_PER_SUB + z * CHUNK, CHUNK)],
                gsem.at[0])
        for z in range(ROWS_PER_SUB // CHUNK):
            pltpu.make_async_copy(
                rows_v.at[0],
                agg_sh.at[pl.ds(sid * ROWS_PER_SUB + z * CHUNK, CHUNK)],
                gsem.at[0]).wait()
        plsc.subcore_barrier()

        def _round(j, first, gi, di):
            for b in range(NBUF):
                def _wait_s(b=b):
                    pltpu.make_async_copy(
                        rows_v.at[b], agg_sh.at[di.at[j + b]],
                        ssem.at[b]).wait()
                if first:
                    pass
                else:
                    _wait_s()
                pltpu.async_copy(table_hbm.at[gi.at[j + b]],
                                 rows_v.at[b], gsem.at[b])
            for b in range(NBUF):
                pltpu.make_async_copy(table_hbm.at[gi.at[j + b]],
                                      rows_v.at[b], gsem.at[b]).wait()
                pltpu.async_copy(rows_v.at[b], agg_sh.at[di.at[j + b]],
                                 ssem.at[b], add=True)

        pltpu.async_copy(gidx_hbm.at[wid, 0], gidx_v.at[0], isem)
        pltpu.async_copy(dst_hbm.at[wid, 0], dst_v.at[0], isem)
        for s in range(SEGS):
            gi = gidx_v.at[s % 2]
            di = dst_v.at[s % 2]
            pltpu.make_async_copy(gidx_hbm.at[wid, s], gi, isem).wait()
            pltpu.make_async_copy(dst_hbm.at[wid, s], di, isem).wait()

            _round(0, s == 0, gi, di)
            if s + 1 < SEGS:
                pltpu.async_copy(gidx_hbm.at[wid, s + 1],
                                 gidx_v.at[(s + 1) % 2], isem)
                pltpu.async_copy(dst_hbm.at[wid, s + 1],
                                 dst_v.at[(s + 1) % 2], isem)

            @pl.loop(NBUF, SEG_CHUNKS, step=NBUF)
            def _(j, gi=gi, di=di):
                _round(j, False, gi, di)

        for b in range(NBUF):
            pltpu.make_async_copy(
                rows_v.at[b],
                agg_sh.at[dst_v.at[(SEGS - 1) % 2].at[SEG_CHUNKS - NBUF + b]],
                ssem.at[b]).wait()
        plsc.subcore_barrier()
        pltpu.sync_copy(
            agg_sh.at[pl.ds(sid * ROWS_PER_SUB, ROWS_PER_SUB)],
            out_hbm.at[cid, pl.ds(sid * ROWS_PER_SUB, ROWS_PER_SUB)])

    return k(table, gidx3, dst3)


def kernel(edge_index, edge_type, emb, W1, W1_loop, b1, W2, W2_loop, b2):
    ei3 = edge_index.astype(jnp.int32).reshape(2, 2500, 128)
    gidx2, dst2 = _gidx(ei3, edge_type.astype(jnp.int32).reshape(2500, 128))
    gidx3 = gidx2.reshape(NW, SEGS, SEG_CHUNKS, CHUNK)
    dst3 = dst2.reshape(NW, SEGS, SEG_CHUNKS, CHUNK)

    bf16 = jnp.bfloat16
    w1_all = jnp.concatenate([W1, W1_loop[None]], axis=0).astype(bf16)
    w2_all = jnp.concatenate([W2, W2_loop[None]], axis=0).astype(bf16)

    t1 = _transform1(emb.astype(bf16), w1_all, b1.reshape(1, D))
    p1 = _scatter(t1, gidx3, dst3)
    t2 = _transform2(p1, t1, w2_all, b2.reshape(1, D))
    p2 = _scatter(t2, gidx3, dst3)
    x2 = _combine(p2, t2)
    return x2


# speedup vs baseline: 1.1575x; 1.0123x over previous
import functools

import jax
import jax.numpy as jnp
from jax import lax
from jax.experimental import pallas as pl
from jax.experimental.pallas import tpu as pltpu
from jax.experimental.pallas import tpu_sc as plsc

N = 10000
R = 8
D = 128
E = 320000

NC, NS = 2, 16
NW = NC * NS
EPW = E // NW
CHUNK = 40
NCHUNK = EPW // CHUNK
SEGS = 25
SEG_CHUNKS = NCHUNK // SEGS
N_PAD = 10240
ROWS_PER_SUB = N_PAD // NS

BN = 2000
NB = N // BN
NBUF = 5


def _gidx_body(ei_ref, t_ref, g_ref, d_ref):
    v = ei_ref[0]
    g_ref[...] = (v // BN) * ((R + 1) * BN) + t_ref[...] * BN + v % BN
    d_ref[...] = ei_ref[1]


def _gidx(ei3, etype2):
    return pl.pallas_call(
        _gidx_body,
        grid=(1,),
        in_specs=[pl.BlockSpec((2, 2500, 128), lambda i: (0, 0, 0)),
                  pl.BlockSpec((2500, 128), lambda i: (0, 0))],
        out_specs=[pl.BlockSpec((2500, 128), lambda i: (0, 0)),
                   pl.BlockSpec((2500, 128), lambda i: (0, 0))],
        out_shape=[jax.ShapeDtypeStruct((2500, 128), jnp.int32),
                   jax.ShapeDtypeStruct((2500, 128), jnp.int32)],
    )(ei3, etype2)


def _mm_all(x16, w_ref, wl_ref, b_ref, o_ref):
    bf16 = jnp.bfloat16
    for r in range(R):
        o_ref[pl.ds(r * BN, BN), :] = jnp.dot(
            x16, w_ref[r].astype(bf16), preferred_element_type=jnp.float32)
    o_ref[pl.ds(R * BN, BN), :] = jnp.dot(
        x16, wl_ref[...].astype(bf16),
        preferred_element_type=jnp.float32) + b_ref[...]


def _transform1_body(h_ref, w_ref, wl_ref, b_ref, o_ref):
    _mm_all(h_ref[...].astype(jnp.bfloat16), w_ref, wl_ref, b_ref, o_ref)


def _transform1(h, w, wl, b):
    return pl.pallas_call(
        _transform1_body,
        grid=(NB,),
        in_specs=[
            pl.BlockSpec((BN, D), lambda n: (n, 0)),
            pl.BlockSpec((R, D, D), lambda n: (0, 0, 0)),
            pl.BlockSpec((D, D), lambda n: (0, 0)),
            pl.BlockSpec((1, D), lambda n: (0, 0)),
        ],
        out_specs=pl.BlockSpec(((R + 1) * BN, D), lambda n: (n, 0)),
        out_shape=jax.ShapeDtypeStruct(((R + 1) * N, D), jnp.float32),
        compiler_params=pltpu.CompilerParams(
            dimension_semantics=("parallel",)),
    )(h, w, wl, b)


def _transform2_body(p_ref, tsl_ref, w_ref, wl_ref, b_ref, o_ref):
    x = p_ref[0] + p_ref[1] + tsl_ref[...]
    _mm_all(x.astype(jnp.bfloat16), w_ref, wl_ref, b_ref, o_ref)


def _transform2(parts, t, w, wl, b):
    return pl.pallas_call(
        _transform2_body,
        grid=(NB,),
        in_specs=[
            pl.BlockSpec((2, BN, D), lambda n: (0, n, 0)),
            pl.BlockSpec((BN, D), lambda n: (n * (R + 1) + R, 0)),
            pl.BlockSpec((R, D, D), lambda n: (0, 0, 0)),
            pl.BlockSpec((D, D), lambda n: (0, 0)),
            pl.BlockSpec((1, D), lambda n: (0, 0)),
        ],
        out_specs=pl.BlockSpec(((R + 1) * BN, D), lambda n: (n, 0)),
        out_shape=jax.ShapeDtypeStruct(((R + 1) * N, D), jnp.float32),
        compiler_params=pltpu.CompilerParams(
            dimension_semantics=("parallel",)),
    )(parts, t, w, wl, b)


def _combine_body(p_ref, t_ref, o_ref):
    o_ref[...] = p_ref[0] + p_ref[1] + t_ref[...]


def _combine(parts, t):
    return pl.pallas_call(
        _combine_body,
        grid=(NB,),
        in_specs=[
            pl.BlockSpec((2, BN, D), lambda n: (0, n, 0)),
            pl.BlockSpec((BN, D), lambda n: (n * (R + 1) + R, 0)),
        ],
        out_specs=pl.BlockSpec((BN, D), lambda n: (n, 0)),
        out_shape=jax.ShapeDtypeStruct((N, D), jnp.float32),
        compiler_params=pltpu.CompilerParams(
            dimension_semantics=("parallel",)),
    )(parts, t)


def _scatter(table, gidx3, dst3):
    @functools.partial(
        pl.kernel,
        mesh=plsc.VectorSubcoreMesh(core_axis_name="c", subcore_axis_name="s"),
        out_type=jax.ShapeDtypeStruct((NC, N_PAD, D), jnp.float32),
        scratch_types=[
            pltpu.VMEM((2, SEG_CHUNKS, CHUNK), jnp.int32),
            pltpu.VMEM((2, SEG_CHUNKS, CHUNK), jnp.int32),
            pltpu.VMEM((NBUF, CHUNK, D), jnp.float32),
            pltpu.VMEM_SHARED((N_PAD, D), jnp.float32),
            pltpu.SemaphoreType.DMA((NBUF,)),
            pltpu.SemaphoreType.DMA((NBUF,)),
            pltpu.SemaphoreType.DMA,
        ],
    )
    def k(table_hbm, gidx_hbm, dst_hbm, out_hbm,
          gidx_v, dst_v, rows_v, agg_sh, gsem, ssem, isem):
        cid = lax.axis_index("c")
        sid = lax.axis_index("s")
        wid = sid * NC + cid

        @pl.loop(0, CHUNK)
        def _(i):
            @pl.loop(0, D, step=16)
            def _(c):
                rows_v[0, i, pl.ds(c, 16)] = jnp.zeros((16,), jnp.float32)

        for z in range(ROWS_PER_SUB // CHUNK):
            pltpu.async_copy(
                rows_v.at[0],
                agg_sh.at[pl.ds(sid * ROWS_PER_SUB + z * CHUNK, CHUNK)],
                gsem.at[0])
        for z in range(ROWS_PER_SUB // CHUNK):
            pltpu.make_async_copy(
                rows_v.at[0],
                agg_sh.at[pl.ds(sid * ROWS_PER_SUB + z * CHUNK, CHUNK)],
                gsem.at[0]).wait()
        plsc.subcore_barrier()

        def _round(j, first, gi, di):
            for b in range(NBUF):
                def _wait_s(b=b):
                    pltpu.make_async_copy(
                        rows_v.at[b], agg_sh.at[di.at[j + b]],
                        ssem.at[b]).wait()
                if first:
                    pass
                else:
                    _wait_s()
                pltpu.async_copy(table_hbm.at[gi.at[j + b]],
                                 rows_v.at[b], gsem.at[b])
            for b in range(NBUF):
                pltpu.make_async_copy(table_hbm.at[gi.at[j + b]],
                                      rows_v.at[b], gsem.at[b]).wait()
                pltpu.async_copy(rows_v.at[b], agg_sh.at[di.at[j + b]],
                                 ssem.at[b], add=True)

        pltpu.async_copy(gidx_hbm.at[wid, 0], gidx_v.at[0], isem)
        pltpu.async_copy(dst_hbm.at[wid, 0], dst_v.at[0], isem)
        for s in range(SEGS):
            gi = gidx_v.at[s % 2]
            di = dst_v.at[s % 2]
            pltpu.make_async_copy(gidx_hbm.at[wid, s], gi, isem).wait()
            pltpu.make_async_copy(dst_hbm.at[wid, s], di, isem).wait()

            _round(0, s == 0, gi, di)
            if s + 1 < SEGS:
                pltpu.async_copy(gidx_hbm.at[wid, s + 1],
                                 gidx_v.at[(s + 1) % 2], isem)
                pltpu.async_copy(dst_hbm.at[wid, s + 1],
                                 dst_v.at[(s + 1) % 2], isem)

            @pl.loop(NBUF, SEG_CHUNKS, step=NBUF)
            def _(j, gi=gi, di=di):
                _round(j, False, gi, di)

        for b in range(NBUF):
            pltpu.make_async_copy(
                rows_v.at[b],
                agg_sh.at[dst_v.at[(SEGS - 1) % 2].at[SEG_CHUNKS - NBUF + b]],
                ssem.at[b]).wait()
        plsc.subcore_barrier()
        pltpu.sync_copy(
            agg_sh.at[pl.ds(sid * ROWS_PER_SUB, ROWS_PER_SUB)],
            out_hbm.at[cid, pl.ds(sid * ROWS_PER_SUB, ROWS_PER_SUB)])

    return k(table, gidx3, dst3)


def kernel(edge_index, edge_type, emb, W1, W1_loop, b1, W2, W2_loop, b2):
    ei3 = edge_index.astype(jnp.int32).reshape(2, 2500, 128)
    gidx2, dst2 = _gidx(ei3, edge_type.astype(jnp.int32).reshape(2500, 128))
    gidx3 = gidx2.reshape(NW, SEGS, SEG_CHUNKS, CHUNK)
    dst3 = dst2.reshape(NW, SEGS, SEG_CHUNKS, CHUNK)

    t1 = _transform1(emb, W1, W1_loop, b1.reshape(1, D))
    p1 = _scatter(t1, gidx3, dst3)
    t2 = _transform2(p1, t1, W2, W2_loop, b2.reshape(1, D))
    p2 = _scatter(t2, gidx3, dst3)
    x2 = _combine(p2, t2)
    return x2
